# baseline (speedup 1.0000x reference)
import functools

import jax
import jax.numpy as jnp
from jax import lax
from jax.experimental import pallas as pl
from jax.experimental.pallas import tpu as pltpu
from jax.experimental.pallas import tpu_sc as plsc

N_NODES = 10000
N_EDGES = 320000
D = 128
NUM_RELS = 64
NUM_BASES = 64

NC = 2
NS = 16
NW = NC * NS
CH = 125
CHUNKS = 80
assert NW * CHUNKS * CH == N_EDGES
ZCH = 16
NZ = N_NODES // ZCH
ZITER = -(-NZ // NS)


def _weights_body(coeff_ref, bases_ref, out_ref):
    out_ref[...] = jnp.dot(coeff_ref[...], bases_ref[...],
                           preferred_element_type=jnp.float32)


def _combine_weights(coeff, bases_flat):
    return pl.pallas_call(
        _weights_body,
        out_shape=jax.ShapeDtypeStruct((NUM_RELS, D * D), jnp.float32),
    )(coeff, bases_flat)


def _project_body(x_ref, w_ref, out_ref):
    out_ref[0] = jnp.dot(x_ref[...], w_ref[0],
                         preferred_element_type=jnp.float32)


def _project_all(x, weight):
    return pl.pallas_call(
        _project_body,
        grid=(NUM_RELS + 1,),
        in_specs=[
            pl.BlockSpec((N_NODES, D), lambda r: (0, 0)),
            pl.BlockSpec((1, D, D), lambda r: (r, 0, 0)),
        ],
        out_specs=pl.BlockSpec((1, N_NODES, D), lambda r: (r, 0, 0)),
        out_shape=jax.ShapeDtypeStruct((NUM_RELS + 1, N_NODES, D),
                                       jnp.float32),
    )(x, weight)


def _sc_body(h_hbm, gidx_hbm, didx_hbm, zeros_hbm, out_hbm,
             gidx_v, didx_v, rows0_v, rows1_v, acc_sh, sem0, sem1):
    cid = lax.axis_index("c")
    sid = lax.axis_index("s")
    wid = cid * NS + sid

    @pl.loop(0, ZITER)
    def _(i):
        c = i * NS + sid

        @pl.when(c < NZ)
        def _():
            pltpu.sync_copy(zeros_hbm, acc_sh.at[pl.ds(c * ZCH, ZCH)])

    pltpu.sync_copy(gidx_hbm.at[wid], gidx_v)
    pltpu.sync_copy(didx_hbm.at[wid], didx_v)
    plsc.subcore_barrier()

    @pl.loop(0, CHUNKS)
    def _(j):
        pltpu.async_copy(h_hbm.at[gidx_v.at[j]], rows0_v, sem0).wait()
        pltpu.sync_copy(rows0_v, acc_sh.at[didx_v.at[j]], add=True)

    plsc.subcore_barrier()

    @pl.loop(0, ZITER)
    def _(i):
        c = i * NS + sid

        @pl.when(c < NZ)
        def _():
            zl = pl.ds(c * ZCH, ZCH)
            pltpu.sync_copy(acc_sh.at[zl], out_hbm.at[cid].at[zl])


def _sc_scatter(h_flat, gidx, didx, zeros):
    mesh = plsc.VectorSubcoreMesh(core_axis_name="c", subcore_axis_name="s")
    return pl.kernel(
        _sc_body,
        out_type=jax.ShapeDtypeStruct((NC, N_NODES, D), jnp.float32),
        mesh=mesh,
        scratch_types=[
            pltpu.VMEM((CHUNKS, CH), jnp.int32),
            pltpu.VMEM((CHUNKS, CH), jnp.int32),
            pltpu.VMEM((CH, D), jnp.float32),
            pltpu.VMEM((CH, D), jnp.float32),
            pltpu.VMEM_SHARED((N_NODES, D), jnp.float32),
            pltpu.SemaphoreType.DMA,
            pltpu.SemaphoreType.DMA,
        ],
    )(h_flat, gidx, didx, zeros)


def _tail_body(acc_ref, res_ref, hb_ref, br_ref, g_ref, b_ref, out_ref):
    agg = acc_ref[0] + acc_ref[1]
    h = jnp.maximum(agg + hb_ref[...], 0.0)
    h = h + jnp.maximum(res_ref[0] + br_ref[...], 0.0)
    mean = jnp.mean(h, axis=0, keepdims=True)
    cent = h - mean
    var = jnp.mean(cent * cent, axis=0, keepdims=True)
    inv = lax.rsqrt(var + 1e-5)
    out_ref[...] = cent * inv * g_ref[...] + b_ref[...]


def _tail(acc, h_all, h_bias, b_res, gamma, beta):
    return pl.pallas_call(
        _tail_body,
        grid=(1,),
        in_specs=[
            pl.BlockSpec((NC, N_NODES, D), lambda i: (0, 0, 0)),
            pl.BlockSpec((1, N_NODES, D), lambda i: (NUM_RELS, 0, 0)),
            pl.BlockSpec((1, D), lambda i: (0, 0)),
            pl.BlockSpec((1, D), lambda i: (0, 0)),
            pl.BlockSpec((1, D), lambda i: (0, 0)),
            pl.BlockSpec((1, D), lambda i: (0, 0)),
        ],
        out_specs=pl.BlockSpec((N_NODES, D), lambda i: (0, 0)),
        out_shape=jax.ShapeDtypeStruct((N_NODES, D), jnp.float32),
    )(acc, h_all, h_bias.reshape(1, D), b_res.reshape(1, D),
      gamma.reshape(1, D), beta.reshape(1, D))


def kernel(node_feats, edge_index, etype, bases, coeff, h_bias, W_res, b_res,
           gamma, beta):
    src = edge_index[0].astype(jnp.int32)
    dst = edge_index[1].astype(jnp.int32)
    et = etype.astype(jnp.int32)

    weight = _combine_weights(coeff, bases.reshape(NUM_BASES, D * D))
    weight = jnp.concatenate([weight, W_res.reshape(1, D * D)])
    h_all = _project_all(node_feats, weight.reshape(NUM_RELS + 1, D, D))
    h_flat = h_all.reshape((NUM_RELS + 1) * N_NODES, D)

    gidx = (et * N_NODES + src).reshape(NW, CHUNKS, CH)
    didx = dst.reshape(NW, CHUNKS, CH)

    zeros = jnp.zeros((ZCH, D), jnp.float32)
    acc = _sc_scatter(h_flat, gidx, didx, zeros)

    return _tail(acc, h_all, h_bias, b_res, gamma, beta)


# speedup vs baseline: 1.0670x; 1.0670x over previous
import functools

import jax
import jax.numpy as jnp
from jax import lax
from jax.experimental import pallas as pl
from jax.experimental.pallas import tpu as pltpu
from jax.experimental.pallas import tpu_sc as plsc

N_NODES = 10000
N_EDGES = 320000
D = 128
NUM_RELS = 64
NUM_BASES = 64

NC = 2
NS = 16
NW = NC * NS
CH = 125
CHUNKS = 80
assert NW * CHUNKS * CH == N_EDGES
ZCH = 16
NZ = N_NODES // ZCH
ZITER = -(-NZ // NS)


def _weights_body(coeff_ref, bases_ref, out_ref):
    out_ref[...] = jnp.dot(coeff_ref[...], bases_ref[...],
                           preferred_element_type=jnp.float32)


def _combine_weights(coeff, bases_flat):
    return pl.pallas_call(
        _weights_body,
        out_shape=jax.ShapeDtypeStruct((NUM_RELS, D * D), jnp.float32),
    )(coeff, bases_flat)


def _project_body(x_ref, w_ref, out_ref):
    out_ref[0] = jnp.dot(x_ref[...], w_ref[0],
                         preferred_element_type=jnp.float32)


def _project_all(x, weight):
    return pl.pallas_call(
        _project_body,
        grid=(NUM_RELS + 1,),
        in_specs=[
            pl.BlockSpec((N_NODES, D), lambda r: (0, 0)),
            pl.BlockSpec((1, D, D), lambda r: (r, 0, 0)),
        ],
        out_specs=pl.BlockSpec((1, N_NODES, D), lambda r: (r, 0, 0)),
        out_shape=jax.ShapeDtypeStruct((NUM_RELS + 1, N_NODES, D),
                                       jnp.float32),
    )(x, weight)


def _sc_body(h_hbm, gidx_hbm, didx_hbm, zeros_hbm, out_hbm,
             gidx_v, didx_v, rows0_v, rows1_v, acc_sh, sem0, sem1):
    cid = lax.axis_index("c")
    sid = lax.axis_index("s")
    wid = cid * NS + sid

    @pl.loop(0, ZITER)
    def _(i):
        c = i * NS + sid

        @pl.when(c < NZ)
        def _():
            zl = pl.ds(c * ZCH, ZCH)
            pltpu.sync_copy(zeros_hbm.at[zl], acc_sh.at[zl])

    pltpu.sync_copy(gidx_hbm.at[wid], gidx_v)
    pltpu.sync_copy(didx_hbm.at[wid], didx_v)
    plsc.subcore_barrier()

    @pl.loop(0, CHUNKS)
    def _(j):
        pltpu.async_copy(h_hbm.at[gidx_v.at[j]], rows0_v, sem0).wait()
        pltpu.sync_copy(rows0_v, acc_sh.at[didx_v.at[j]], add=True)

    plsc.subcore_barrier()

    @pl.loop(0, ZITER)
    def _(i):
        c = i * NS + sid

        @pl.when(c < NZ)
        def _():
            zl = pl.ds(c * ZCH, ZCH)
            pltpu.sync_copy(acc_sh.at[zl], out_hbm.at[cid].at[zl])


def _sc_scatter(h_flat, gidx, didx, zeros):
    mesh = plsc.VectorSubcoreMesh(core_axis_name="c", subcore_axis_name="s")
    return pl.kernel(
        _sc_body,
        out_type=jax.ShapeDtypeStruct((NC, N_NODES, D), jnp.float32),
        mesh=mesh,
        scratch_types=[
            pltpu.VMEM((CHUNKS, CH), jnp.int32),
            pltpu.VMEM((CHUNKS, CH), jnp.int32),
            pltpu.VMEM((CH, D), jnp.float32),
            pltpu.VMEM((CH, D), jnp.float32),
            pltpu.VMEM_SHARED((N_NODES, D), jnp.float32),
            pltpu.SemaphoreType.DMA,
            pltpu.SemaphoreType.DMA,
        ],
    )(h_flat, gidx, didx, zeros)


def _tail_body(acc_ref, res_ref, hb_ref, br_ref, g_ref, b_ref, out_ref):
    agg = acc_ref[0] + acc_ref[1]
    h = jnp.maximum(agg + hb_ref[...], 0.0)
    h = h + jnp.maximum(res_ref[0] + br_ref[...], 0.0)
    mean = jnp.mean(h, axis=0, keepdims=True)
    cent = h - mean
    var = jnp.mean(cent * cent, axis=0, keepdims=True)
    inv = lax.rsqrt(var + 1e-5)
    out_ref[...] = cent * inv * g_ref[...] + b_ref[...]


def _tail(acc, h_all, h_bias, b_res, gamma, beta):
    return pl.pallas_call(
        _tail_body,
        grid=(1,),
        in_specs=[
            pl.BlockSpec((NC, N_NODES, D), lambda i: (0, 0, 0)),
            pl.BlockSpec((1, N_NODES, D), lambda i: (NUM_RELS, 0, 0)),
            pl.BlockSpec((1, D), lambda i: (0, 0)),
            pl.BlockSpec((1, D), lambda i: (0, 0)),
            pl.BlockSpec((1, D), lambda i: (0, 0)),
            pl.BlockSpec((1, D), lambda i: (0, 0)),
        ],
        out_specs=pl.BlockSpec((N_NODES, D), lambda i: (0, 0)),
        out_shape=jax.ShapeDtypeStruct((N_NODES, D), jnp.float32),
    )(acc, h_all, h_bias.reshape(1, D), b_res.reshape(1, D),
      gamma.reshape(1, D), beta.reshape(1, D))


def kernel(node_feats, edge_index, etype, bases, coeff, h_bias, W_res, b_res,
           gamma, beta):
    src = edge_index[0].astype(jnp.int32)
    dst = edge_index[1].astype(jnp.int32)
    et = etype.astype(jnp.int32)

    weight = _combine_weights(coeff, bases.reshape(NUM_BASES, D * D))
    weight = jnp.concatenate([weight, W_res.reshape(1, D * D)])
    h_all = _project_all(node_feats, weight.reshape(NUM_RELS + 1, D, D))
    h_flat = h_all.reshape((NUM_RELS + 1) * N_NODES, D)

    gidx = (et * N_NODES + src).reshape(NW, CHUNKS, CH)
    didx = dst.reshape(NW, CHUNKS, CH)

    zeros = jnp.zeros((N_NODES, D), jnp.float32)
    acc = _sc_scatter(h_flat, gidx, didx, zeros)

    return _tail(acc, h_all, h_bias, b_res, gamma, beta)


# speedup vs baseline: 1.0705x; 1.0033x over previous
import functools

import jax
import jax.numpy as jnp
from jax import lax
from jax.experimental import pallas as pl
from jax.experimental.pallas import tpu as pltpu
from jax.experimental.pallas import tpu_sc as plsc

N_NODES = 10000
N_EDGES = 320000
D = 128
NUM_RELS = 64
NUM_BASES = 64

NC = 2
NS = 16
NW = NC * NS
CH = 125
CHUNKS = 80
assert NW * CHUNKS * CH == N_EDGES
ZCH = 16
NZ = N_NODES // ZCH
ZITER = -(-NZ // NS)


def _weights_body(coeff_ref, bases_ref, out_ref):
    out_ref[...] = jnp.dot(coeff_ref[...], bases_ref[...],
                           preferred_element_type=jnp.float32)


def _combine_weights(coeff, bases_flat):
    return pl.pallas_call(
        _weights_body,
        out_shape=jax.ShapeDtypeStruct((NUM_RELS, D * D), jnp.float32),
    )(coeff, bases_flat)


def _project_body(x_ref, w_ref, out_ref):
    out_ref[0] = jnp.dot(x_ref[...], w_ref[0],
                         preferred_element_type=jnp.float32)


def _project_all(x, weight):
    return pl.pallas_call(
        _project_body,
        grid=(NUM_RELS,),
        in_specs=[
            pl.BlockSpec((N_NODES, D), lambda r: (0, 0)),
            pl.BlockSpec((1, D, D), lambda r: (r, 0, 0)),
        ],
        out_specs=pl.BlockSpec((1, N_NODES, D), lambda r: (r, 0, 0)),
        out_shape=jax.ShapeDtypeStruct((NUM_RELS, N_NODES, D), jnp.float32),
    )(x, weight)


def _sc_body(h_hbm, gidx_hbm, didx_hbm, zeros_hbm, out_hbm,
             gidx_v, didx_v, rows0_v, rows1_v, acc_sh, sem0, sem1):
    cid = lax.axis_index("c")
    sid = lax.axis_index("s")
    wid = cid * NS + sid

    @pl.loop(0, ZITER)
    def _(i):
        c = i * NS + sid

        @pl.when(c < NZ)
        def _():
            zl = pl.ds(c * ZCH, ZCH)
            pltpu.sync_copy(zeros_hbm.at[zl], acc_sh.at[zl])

    pltpu.sync_copy(gidx_hbm.at[wid], gidx_v)
    pltpu.sync_copy(didx_hbm.at[wid], didx_v)
    plsc.subcore_barrier()

    @pl.loop(0, CHUNKS)
    def _(j):
        pltpu.async_copy(h_hbm.at[gidx_v.at[j]], rows0_v, sem0).wait()
        pltpu.sync_copy(rows0_v, acc_sh.at[didx_v.at[j]], add=True)

    plsc.subcore_barrier()

    @pl.loop(0, ZITER)
    def _(i):
        c = i * NS + sid

        @pl.when(c < NZ)
        def _():
            zl = pl.ds(c * ZCH, ZCH)
            pltpu.sync_copy(acc_sh.at[zl], out_hbm.at[cid].at[zl])


def _sc_scatter(h_flat, gidx, didx, zeros):
    mesh = plsc.VectorSubcoreMesh(core_axis_name="c", subcore_axis_name="s")
    return pl.kernel(
        _sc_body,
        out_type=jax.ShapeDtypeStruct((NC, N_NODES, D), jnp.float32),
        mesh=mesh,
        scratch_types=[
            pltpu.VMEM((CHUNKS, CH), jnp.int32),
            pltpu.VMEM((CHUNKS, CH), jnp.int32),
            pltpu.VMEM((CH, D), jnp.float32),
            pltpu.VMEM((CH, D), jnp.float32),
            pltpu.VMEM_SHARED((N_NODES, D), jnp.float32),
            pltpu.SemaphoreType.DMA,
            pltpu.SemaphoreType.DMA,
        ],
    )(h_flat, gidx, didx, zeros)


def _tail_body(acc_ref, x_ref, hb_ref, wr_ref, br_ref, g_ref, b_ref,
               out_ref):
    agg = acc_ref[0] + acc_ref[1]
    h = jnp.maximum(agg + hb_ref[...], 0.0)
    res = jnp.dot(x_ref[...], wr_ref[...], preferred_element_type=jnp.float32)
    h = h + jnp.maximum(res + br_ref[...], 0.0)
    mean = jnp.mean(h, axis=0, keepdims=True)
    cent = h - mean
    var = jnp.mean(cent * cent, axis=0, keepdims=True)
    inv = lax.rsqrt(var + 1e-5)
    out_ref[...] = cent * inv * g_ref[...] + b_ref[...]


def _tail(acc, x, h_bias, W_res, b_res, gamma, beta):
    return pl.pallas_call(
        _tail_body,
        out_shape=jax.ShapeDtypeStruct((N_NODES, D), jnp.float32),
    )(acc, x, h_bias.reshape(1, D), W_res, b_res.reshape(1, D),
      gamma.reshape(1, D), beta.reshape(1, D))


def kernel(node_feats, edge_index, etype, bases, coeff, h_bias, W_res, b_res,
           gamma, beta):
    src = edge_index[0].astype(jnp.int32)
    dst = edge_index[1].astype(jnp.int32)
    et = etype.astype(jnp.int32)

    weight = _combine_weights(coeff, bases.reshape(NUM_BASES, D * D))
    h_all = _project_all(node_feats, weight.reshape(NUM_RELS, D, D))
    h_flat = h_all.reshape(NUM_RELS * N_NODES, D)

    gidx = (et * N_NODES + src).reshape(NW, CHUNKS, CH)
    didx = dst.reshape(NW, CHUNKS, CH)

    zeros = jnp.zeros((N_NODES, D), jnp.float32)
    acc = _sc_scatter(h_flat, gidx, didx, zeros)

    return _tail(acc, node_feats, h_bias, W_res, b_res, gamma, beta)


# speedup vs baseline: 1.0719x; 1.0013x over previous
import jax
import jax.numpy as jnp
from jax import lax
from jax.experimental import pallas as pl
from jax.experimental.pallas import tpu as pltpu
from jax.experimental.pallas import tpu_sc as plsc

N_NODES = 10000
N_EDGES = 320000
D = 128
NUM_RELS = 64
NUM_BASES = 64

NC = 2
NS = 16
NW = NC * NS
CH = 125
CHUNKS = 80
assert NW * CHUNKS * CH == N_EDGES
ZCH = 16
NZ = N_NODES // ZCH
ZITER = -(-NZ // NS)


def _weights_body(coeff_ref, bases_ref, out_ref):
    out_ref[...] = jnp.dot(coeff_ref[...], bases_ref[...],
                           preferred_element_type=jnp.float32)


def _combine_weights(coeff, bases_flat):
    return pl.pallas_call(
        _weights_body,
        out_shape=jax.ShapeDtypeStruct((NUM_RELS, D * D), jnp.float32),
    )(coeff, bases_flat)


def _project_body(x_ref, w_ref, out_ref):
    out_ref[0] = jnp.dot(x_ref[...], w_ref[0],
                         preferred_element_type=jnp.float32)


def _project_all(x, weight):
    return pl.pallas_call(
        _project_body,
        grid=(NUM_RELS,),
        in_specs=[
            pl.BlockSpec((N_NODES, D), lambda r: (0, 0)),
            pl.BlockSpec((1, D, D), lambda r: (r, 0, 0)),
        ],
        out_specs=pl.BlockSpec((1, N_NODES, D), lambda r: (r, 0, 0)),
        out_shape=jax.ShapeDtypeStruct((NUM_RELS, N_NODES, D), jnp.float32),
    )(x, weight)


def _sc_body(h_hbm, gidx_hbm, didx_hbm, zeros_hbm, out_hbm,
             gidx_v, didx_v, rows_v, acc_sh, sem):
    cid = lax.axis_index("c")
    sid = lax.axis_index("s")
    wid = cid * NS + sid

    @pl.loop(0, ZITER)
    def _(i):
        c = i * NS + sid

        @pl.when(c < NZ)
        def _():
            zl = pl.ds(c * ZCH, ZCH)
            pltpu.sync_copy(zeros_hbm.at[zl], acc_sh.at[zl])

    pltpu.sync_copy(gidx_hbm.at[wid], gidx_v)
    pltpu.sync_copy(didx_hbm.at[wid], didx_v)
    plsc.subcore_barrier()

    @pl.loop(0, CHUNKS)
    def _(j):
        pltpu.async_copy(h_hbm.at[gidx_v.at[j]], rows_v, sem).wait()
        pltpu.sync_copy(rows_v, acc_sh.at[didx_v.at[j]], add=True)

    plsc.subcore_barrier()

    @pl.loop(0, ZITER)
    def _(i):
        c = i * NS + sid

        @pl.when(c < NZ)
        def _():
            zl = pl.ds(c * ZCH, ZCH)
            pltpu.sync_copy(acc_sh.at[zl], out_hbm.at[cid].at[zl])


def _sc_scatter(h_flat, gidx, didx, zeros):
    mesh = plsc.VectorSubcoreMesh(core_axis_name="c", subcore_axis_name="s")
    return pl.kernel(
        _sc_body,
        out_type=jax.ShapeDtypeStruct((NC, N_NODES, D), jnp.float32),
        mesh=mesh,
        scratch_types=[
            pltpu.VMEM((CHUNKS, CH), jnp.int32),
            pltpu.VMEM((CHUNKS, CH), jnp.int32),
            pltpu.VMEM((CH, D), jnp.float32),
            pltpu.VMEM_SHARED((N_NODES, D), jnp.float32),
            pltpu.SemaphoreType.DMA,
        ],
    )(h_flat, gidx, didx, zeros)


def _tail_body(acc_ref, x_ref, hb_ref, wr_ref, br_ref, g_ref, b_ref,
               out_ref):
    agg = acc_ref[0] + acc_ref[1]
    h = jnp.maximum(agg + hb_ref[...], 0.0)
    res = jnp.dot(x_ref[...], wr_ref[...], preferred_element_type=jnp.float32)
    h = h + jnp.maximum(res + br_ref[...], 0.0)
    mean = jnp.mean(h, axis=0, keepdims=True)
    cent = h - mean
    var = jnp.mean(cent * cent, axis=0, keepdims=True)
    inv = lax.rsqrt(var + 1e-5)
    out_ref[...] = cent * inv * g_ref[...] + b_ref[...]


def _tail(acc, x, h_bias, W_res, b_res, gamma, beta):
    return pl.pallas_call(
        _tail_body,
        out_shape=jax.ShapeDtypeStruct((N_NODES, D), jnp.float32),
    )(acc, x, h_bias.reshape(1, D), W_res, b_res.reshape(1, D),
      gamma.reshape(1, D), beta.reshape(1, D))


def kernel(node_feats, edge_index, etype, bases, coeff, h_bias, W_res, b_res,
           gamma, beta):
    src = edge_index[0].astype(jnp.int32)
    dst = edge_index[1].astype(jnp.int32)
    et = etype.astype(jnp.int32)

    weight = _combine_weights(coeff, bases.reshape(NUM_BASES, D * D))
    h_all = _project_all(node_feats, weight.reshape(NUM_RELS, D, D))
    h_flat = h_all.reshape(NUM_RELS * N_NODES, D)

    gidx = (et * N_NODES + src).reshape(NW, CHUNKS, CH)
    didx = dst.reshape(NW, CHUNKS, CH)

    zeros = jnp.zeros((N_NODES, D), jnp.float32)
    acc = _sc_scatter(h_flat, gidx, didx, zeros)

    return _tail(acc, node_feats, h_bias, W_res, b_res, gamma, beta)
